# baseline (speedup 1.0000x reference)
import functools

import jax
import jax.numpy as jnp
from jax import lax
from jax.experimental import pallas as pl
from jax.experimental.pallas import tpu as pltpu
from jax.experimental.pallas import tpu_sc as plsc

N_NODES = 10000
N_EDGES = 320000
D = 128
DG = 8
NC = 2
NS = 16
NW = NC * NS
EPT = N_EDGES // NW
K = 40
CHUNKS = EPT // K
BLOCKS = 2
BCH = CHUNKS // BLOCKS
RING = 6
HALF = 4
GROUPS = (BCH - RING) // RING
PEEL = (BCH - RING) % RING
ROWS_PT = 640
N_PAD = NS * ROWS_PT


def _sc_aggregate(x, ei2, zrows, zdeg, ones8):
    mesh = plsc.VectorSubcoreMesh(core_axis_name="c", subcore_axis_name="s")

    @functools.partial(
        pl.kernel,
        out_type=(jax.ShapeDtypeStruct((NC, N_PAD, D), jnp.float32),
                  jax.ShapeDtypeStruct((NC, N_PAD, DG), jnp.float32)),
        mesh=mesh,
        scratch_types=[
            pltpu.VMEM((BCH, K), jnp.int32),
            pltpu.VMEM((BCH, K), jnp.int32),
            pltpu.VMEM((RING, K, D), jnp.float32),
            pltpu.VMEM((K, DG), jnp.float32),
            pltpu.VMEM_SHARED((N_PAD, D), jnp.float32),
            pltpu.VMEM_SHARED((N_PAD, DG), jnp.float32),
            pltpu.SemaphoreType.DMA((RING,)),
            pltpu.SemaphoreType.DMA((RING,)),
            pltpu.SemaphoreType.DMA((RING,)),
        ],
        compiler_params=pltpu.CompilerParams(use_tc_tiling_on_sc=False),
    )
    def body(x_hbm, ei_hbm, zrows_hbm, zdeg_hbm, ones8_hbm,
             out_hbm, outd_hbm,
             src_v, dst_v, rows, ones_v, agg_sh, deg_sh, gsem, ssem, dsem):
        cid = lax.axis_index("c")
        sid = lax.axis_index("s")
        wid = cid * NS + sid
        crow = wid * CHUNKS

        def gather_start(j, b):
            pltpu.async_copy(x_hbm.at[src_v.at[j]], rows.at[b], gsem.at[b])

        def gather_wait(j, b):
            pltpu.make_async_copy(
                x_hbm.at[src_v.at[j]], rows.at[b], gsem.at[b]).wait()

        def scatter_start(j, b):
            pltpu.async_copy(rows.at[b], agg_sh.at[dst_v.at[j]],
                             ssem.at[b], add=True)
            pltpu.async_copy(ones_v, deg_sh.at[dst_v.at[j]],
                             dsem.at[b], add=True)

        def scatter_wait(j, b):
            pltpu.make_async_copy(
                rows.at[b], agg_sh.at[dst_v.at[j]], ssem.at[b]).wait()
            pltpu.make_async_copy(
                ones_v, deg_sh.at[dst_v.at[j]], dsem.at[b]).wait()

        rbase = sid * ROWS_PT
        c0 = pltpu.async_copy(ones8_hbm, ones_v, gsem.at[0])
        c1 = pltpu.async_copy(
            zrows_hbm, agg_sh.at[pl.ds(rbase, ROWS_PT), :], gsem.at[1])
        c2 = pltpu.async_copy(
            zdeg_hbm, deg_sh.at[pl.ds(rbase, ROWS_PT), :], gsem.at[2])
        c3 = pltpu.async_copy(
            ei_hbm.at[0, pl.ds(crow, BCH), :], src_v, gsem.at[3])
        c4 = pltpu.async_copy(
            ei_hbm.at[1, pl.ds(crow, BCH), :], dst_v, gsem.at[4])
        c0.wait(); c1.wait(); c2.wait(); c3.wait(); c4.wait()
        plsc.subcore_barrier()

        for h in range(BLOCKS):
            if h > 0:
                pltpu.sync_copy(
                    ei_hbm.at[0, pl.ds(crow + h * BCH, BCH), :], src_v)
                pltpu.sync_copy(
                    ei_hbm.at[1, pl.ds(crow + h * BCH, BCH), :], dst_v)

            for b in range(HALF):
                gather_start(b, b)
            for b in range(HALF, RING):
                gather_start(b, b)
                gather_wait(b - HALF, b - HALF)
                scatter_start(b - HALF, b - HALF)

            def group(g, _):
                j0 = RING + g * RING
                for b in range(RING):
                    j = j0 + b
                    scatter_wait(j - RING, b)
                    gather_start(j, b)
                    bm = (b - HALF) % RING
                    gather_wait(j - HALF, bm)
                    scatter_start(j - HALF, bm)
                return 0

            lax.fori_loop(0, GROUPS, group, 0)

            for i in range(PEEL):
                j = RING + GROUPS * RING + i
                b = j % RING
                scatter_wait(j - RING, b)
                gather_start(j, b)
                bm = (b - HALF) % RING
                gather_wait(j - HALF, bm)
                scatter_start(j - HALF, bm)

            for i in range(HALF):
                j = BCH - HALF + i
                b = j % RING
                gather_wait(j, b)
                scatter_start(j, b)
            for b in range(RING):
                jj = BCH - RING + b
                scatter_wait(jj, jj % RING)

        plsc.subcore_barrier()

        pltpu.sync_copy(agg_sh.at[pl.ds(rbase, ROWS_PT), :],
                        out_hbm.at[cid, pl.ds(rbase, ROWS_PT), :])
        pltpu.sync_copy(deg_sh.at[pl.ds(rbase, ROWS_PT), :],
                        outd_hbm.at[cid, pl.ds(rbase, ROWS_PT), :])

    return body(x, ei2, zrows, zdeg, ones8)


def _tc_xw1(x, W, b2d):
    R = 2000
    grid = (N_NODES // R,)

    def body(x_ref, w_ref, b_ref, o_ref):
        o_ref[...] = jnp.dot(
            x_ref[...], w_ref[:D], preferred_element_type=jnp.float32
        ) + b_ref[...]

    return pl.pallas_call(
        body,
        grid=grid,
        in_specs=[
            pl.BlockSpec((R, D), lambda i: (i, 0)),
            pl.BlockSpec((2 * D, D), lambda i: (0, 0)),
            pl.BlockSpec((1, D), lambda i: (0, 0)),
        ],
        out_specs=pl.BlockSpec((R, D), lambda i: (i, 0)),
        out_shape=jax.ShapeDtypeStruct((N_NODES, D), jnp.float32),
    )(x, W, b2d)


def _tc_combine(xw, part, dpart, W):
    R = 5000
    grid = (N_NODES // R,)

    def body(xw_ref, p_ref, d_ref, w_ref, o_ref):
        acc = p_ref[0] + p_ref[1]
        dsum = d_ref[0] + d_ref[1]
        deg = jnp.maximum(dsum[:, 0:1], 1.0)
        mean = acc / deg
        o_ref[...] = xw_ref[...] + jnp.dot(
            mean, w_ref[D:], preferred_element_type=jnp.float32)

    return pl.pallas_call(
        body,
        grid=grid,
        in_specs=[
            pl.BlockSpec((R, D), lambda i: (i, 0)),
            pl.BlockSpec((2, R, D), lambda i: (0, i, 0)),
            pl.BlockSpec((2, R, DG), lambda i: (0, i, 0)),
            pl.BlockSpec((2 * D, D), lambda i: (0, 0)),
        ],
        out_specs=pl.BlockSpec((R, D), lambda i: (i, 0)),
        out_shape=jax.ShapeDtypeStruct((N_NODES, D), jnp.float32),
    )(xw, part, dpart, W)


def kernel(x, edge_index, W, b):
    ei2 = edge_index.astype(jnp.int32).reshape(2, N_EDGES // K, K)
    zrows = jnp.zeros((ROWS_PT, D), jnp.float32)
    zdeg = jnp.zeros((ROWS_PT, DG), jnp.float32)
    ones8 = jnp.zeros((K, DG), jnp.float32).at[:, 0].set(1.0)
    xw = _tc_xw1(x, W, b.reshape(1, D))
    part, dpart = _sc_aggregate(x, ei2, zrows, zdeg, ones8)
    return _tc_combine(xw, part, dpart, W)


# speedup vs baseline: 1.0001x; 1.0001x over previous
import functools

import jax
import jax.numpy as jnp
from jax import lax
from jax.experimental import pallas as pl
from jax.experimental.pallas import tpu as pltpu
from jax.experimental.pallas import tpu_sc as plsc

N_NODES = 10000
N_EDGES = 320000
D = 128
DG = 8
NC = 2
NS = 16
NW = NC * NS
EPT = N_EDGES // NW
K = 40
CHUNKS = EPT // K
BLOCKS = 2
BCH = CHUNKS // BLOCKS
RING = 6
HALF = 5
GROUPS = (BCH - RING) // RING
PEEL = (BCH - RING) % RING
ROWS_PT = 640
N_PAD = NS * ROWS_PT


def _sc_aggregate(x, ei2, zrows, zdeg, ones8):
    mesh = plsc.VectorSubcoreMesh(core_axis_name="c", subcore_axis_name="s")

    @functools.partial(
        pl.kernel,
        out_type=(jax.ShapeDtypeStruct((NC, N_PAD, D), jnp.float32),
                  jax.ShapeDtypeStruct((NC, N_PAD, DG), jnp.float32)),
        mesh=mesh,
        scratch_types=[
            pltpu.VMEM((BCH, K), jnp.int32),
            pltpu.VMEM((BCH, K), jnp.int32),
            pltpu.VMEM((RING, K, D), jnp.float32),
            pltpu.VMEM((K, DG), jnp.float32),
            pltpu.VMEM_SHARED((N_PAD, D), jnp.float32),
            pltpu.VMEM_SHARED((N_PAD, DG), jnp.float32),
            pltpu.SemaphoreType.DMA((RING,)),
            pltpu.SemaphoreType.DMA((RING,)),
            pltpu.SemaphoreType.DMA((RING,)),
        ],
        compiler_params=pltpu.CompilerParams(use_tc_tiling_on_sc=False),
    )
    def body(x_hbm, ei_hbm, zrows_hbm, zdeg_hbm, ones8_hbm,
             out_hbm, outd_hbm,
             src_v, dst_v, rows, ones_v, agg_sh, deg_sh, gsem, ssem, dsem):
        cid = lax.axis_index("c")
        sid = lax.axis_index("s")
        wid = cid * NS + sid
        crow = wid * CHUNKS

        def gather_start(j, b):
            pltpu.async_copy(x_hbm.at[src_v.at[j]], rows.at[b], gsem.at[b])

        def gather_wait(j, b):
            pltpu.make_async_copy(
                x_hbm.at[src_v.at[j]], rows.at[b], gsem.at[b]).wait()

        def scatter_start(j, b):
            pltpu.async_copy(rows.at[b], agg_sh.at[dst_v.at[j]],
                             ssem.at[b], add=True)
            pltpu.async_copy(ones_v, deg_sh.at[dst_v.at[j]],
                             dsem.at[b], add=True)

        def scatter_wait(j, b):
            pltpu.make_async_copy(
                rows.at[b], agg_sh.at[dst_v.at[j]], ssem.at[b]).wait()
            pltpu.make_async_copy(
                ones_v, deg_sh.at[dst_v.at[j]], dsem.at[b]).wait()

        rbase = sid * ROWS_PT
        c0 = pltpu.async_copy(ones8_hbm, ones_v, gsem.at[0])
        c1 = pltpu.async_copy(
            zrows_hbm, agg_sh.at[pl.ds(rbase, ROWS_PT), :], gsem.at[1])
        c2 = pltpu.async_copy(
            zdeg_hbm, deg_sh.at[pl.ds(rbase, ROWS_PT), :], gsem.at[2])
        c3 = pltpu.async_copy(
            ei_hbm.at[0, pl.ds(crow, BCH), :], src_v, gsem.at[3])
        c4 = pltpu.async_copy(
            ei_hbm.at[1, pl.ds(crow, BCH), :], dst_v, gsem.at[4])
        c0.wait(); c1.wait(); c2.wait(); c3.wait(); c4.wait()
        plsc.subcore_barrier()

        for h in range(BLOCKS):
            if h > 0:
                pltpu.sync_copy(
                    ei_hbm.at[0, pl.ds(crow + h * BCH, BCH), :], src_v)
                pltpu.sync_copy(
                    ei_hbm.at[1, pl.ds(crow + h * BCH, BCH), :], dst_v)

            for b in range(HALF):
                gather_start(b, b)
            for b in range(HALF, RING):
                gather_start(b, b)
                gather_wait(b - HALF, b - HALF)
                scatter_start(b - HALF, b - HALF)

            def group(g, _):
                j0 = RING + g * RING
                for b in range(RING):
                    j = j0 + b
                    scatter_wait(j - RING, b)
                    gather_start(j, b)
                    bm = (b - HALF) % RING
                    gather_wait(j - HALF, bm)
                    scatter_start(j - HALF, bm)
                return 0

            lax.fori_loop(0, GROUPS, group, 0)

            for i in range(PEEL):
                j = RING + GROUPS * RING + i
                b = j % RING
                scatter_wait(j - RING, b)
                gather_start(j, b)
                bm = (b - HALF) % RING
                gather_wait(j - HALF, bm)
                scatter_start(j - HALF, bm)

            for i in range(HALF):
                j = BCH - HALF + i
                b = j % RING
                gather_wait(j, b)
                scatter_start(j, b)
            for b in range(RING):
                jj = BCH - RING + b
                scatter_wait(jj, jj % RING)

        plsc.subcore_barrier()

        pltpu.sync_copy(agg_sh.at[pl.ds(rbase, ROWS_PT), :],
                        out_hbm.at[cid, pl.ds(rbase, ROWS_PT), :])
        pltpu.sync_copy(deg_sh.at[pl.ds(rbase, ROWS_PT), :],
                        outd_hbm.at[cid, pl.ds(rbase, ROWS_PT), :])

    return body(x, ei2, zrows, zdeg, ones8)


def _tc_xw1(x, W, b2d):
    R = 2000
    grid = (N_NODES // R,)

    def body(x_ref, w_ref, b_ref, o_ref):
        o_ref[...] = jnp.dot(
            x_ref[...], w_ref[:D], preferred_element_type=jnp.float32
        ) + b_ref[...]

    return pl.pallas_call(
        body,
        grid=grid,
        in_specs=[
            pl.BlockSpec((R, D), lambda i: (i, 0)),
            pl.BlockSpec((2 * D, D), lambda i: (0, 0)),
            pl.BlockSpec((1, D), lambda i: (0, 0)),
        ],
        out_specs=pl.BlockSpec((R, D), lambda i: (i, 0)),
        out_shape=jax.ShapeDtypeStruct((N_NODES, D), jnp.float32),
    )(x, W, b2d)


def _tc_combine(xw, part, dpart, W):
    R = 2000
    grid = (N_NODES // R,)

    def body(xw_ref, p_ref, d_ref, w_ref, o_ref):
        acc = p_ref[0] + p_ref[1]
        dsum = d_ref[0] + d_ref[1]
        deg = jnp.maximum(dsum[:, 0:1], 1.0)
        mean = acc / deg
        o_ref[...] = xw_ref[...] + jnp.dot(
            mean, w_ref[D:], preferred_element_type=jnp.float32)

    return pl.pallas_call(
        body,
        grid=grid,
        in_specs=[
            pl.BlockSpec((R, D), lambda i: (i, 0)),
            pl.BlockSpec((2, R, D), lambda i: (0, i, 0)),
            pl.BlockSpec((2, R, DG), lambda i: (0, i, 0)),
            pl.BlockSpec((2 * D, D), lambda i: (0, 0)),
        ],
        out_specs=pl.BlockSpec((R, D), lambda i: (i, 0)),
        out_shape=jax.ShapeDtypeStruct((N_NODES, D), jnp.float32),
    )(xw, part, dpart, W)


def kernel(x, edge_index, W, b):
    ei2 = edge_index.astype(jnp.int32).reshape(2, N_EDGES // K, K)
    zrows = jnp.zeros((ROWS_PT, D), jnp.float32)
    zdeg = jnp.zeros((ROWS_PT, DG), jnp.float32)
    ones8 = jnp.zeros((K, DG), jnp.float32).at[:, 0].set(1.0)
    xw = _tc_xw1(x, W, b.reshape(1, D))
    part, dpart = _sc_aggregate(x, ei2, zrows, zdeg, ones8)
    return _tc_combine(xw, part, dpart, W)


# speedup vs baseline: 1.0028x; 1.0027x over previous
import functools

import jax
import jax.numpy as jnp
from jax import lax
from jax.experimental import pallas as pl
from jax.experimental.pallas import tpu as pltpu
from jax.experimental.pallas import tpu_sc as plsc

N_NODES = 10000
N_EDGES = 320000
D = 128
DG = 8
NC = 2
NS = 16
NW = NC * NS
EPT = N_EDGES // NW
K = 40
CHUNKS = EPT // K
BLOCKS = 2
BCH = CHUNKS // BLOCKS
RING = 6
HALF = 4
GROUPS = (BCH - RING) // RING
PEEL = (BCH - RING) % RING
ROWS_PT = 640
N_PAD = NS * ROWS_PT


def _sc_aggregate(x, ei2, zrows, zdeg, ones8):
    mesh = plsc.VectorSubcoreMesh(core_axis_name="c", subcore_axis_name="s")

    @functools.partial(
        pl.kernel,
        out_type=(jax.ShapeDtypeStruct((NC, N_PAD, D), jnp.float32),
                  jax.ShapeDtypeStruct((NC, N_PAD, DG), jnp.float32)),
        mesh=mesh,
        scratch_types=[
            pltpu.VMEM((BCH, K), jnp.int32),
            pltpu.VMEM((BCH, K), jnp.int32),
            pltpu.VMEM((RING, K, D), jnp.float32),
            pltpu.VMEM((K, DG), jnp.float32),
            pltpu.VMEM_SHARED((N_PAD, D), jnp.float32),
            pltpu.VMEM_SHARED((N_PAD, DG), jnp.float32),
            pltpu.SemaphoreType.DMA((RING,)),
            pltpu.SemaphoreType.DMA((RING,)),
            pltpu.SemaphoreType.DMA((RING,)),
        ],
        compiler_params=pltpu.CompilerParams(use_tc_tiling_on_sc=False),
    )
    def body(x_hbm, ei_hbm, zrows_hbm, zdeg_hbm, ones8_hbm,
             out_hbm, outd_hbm,
             src_v, dst_v, rows, ones_v, agg_sh, deg_sh, gsem, ssem, dsem):
        cid = lax.axis_index("c")
        sid = lax.axis_index("s")
        wid = cid * NS + sid
        crow = wid * CHUNKS

        def gather_start(j, b):
            pltpu.async_copy(x_hbm.at[src_v.at[j]], rows.at[b], gsem.at[b])

        def gather_wait(j, b):
            pltpu.make_async_copy(
                x_hbm.at[src_v.at[j]], rows.at[b], gsem.at[b]).wait()

        def scatter_start(j, b):
            pltpu.async_copy(rows.at[b], agg_sh.at[dst_v.at[j]],
                             ssem.at[b], add=True)
            pltpu.async_copy(ones_v, deg_sh.at[dst_v.at[j]],
                             dsem.at[b], add=True)

        def scatter_wait(j, b):
            pltpu.make_async_copy(
                rows.at[b], agg_sh.at[dst_v.at[j]], ssem.at[b]).wait()
            pltpu.make_async_copy(
                ones_v, deg_sh.at[dst_v.at[j]], dsem.at[b]).wait()

        rbase = sid * ROWS_PT
        c0 = pltpu.async_copy(ones8_hbm, ones_v, gsem.at[0])
        c1 = pltpu.async_copy(
            zrows_hbm, agg_sh.at[pl.ds(rbase, ROWS_PT), :], gsem.at[1])
        c2 = pltpu.async_copy(
            zdeg_hbm, deg_sh.at[pl.ds(rbase, ROWS_PT), :], gsem.at[2])
        c3 = pltpu.async_copy(
            ei_hbm.at[0, pl.ds(crow, BCH), :], src_v, gsem.at[3])
        c4 = pltpu.async_copy(
            ei_hbm.at[1, pl.ds(crow, BCH), :], dst_v, gsem.at[4])
        c0.wait(); c1.wait(); c2.wait(); c3.wait(); c4.wait()
        plsc.subcore_barrier()

        for h in range(BLOCKS):
            if h > 0:
                pltpu.sync_copy(
                    ei_hbm.at[0, pl.ds(crow + h * BCH, BCH), :], src_v)
                pltpu.sync_copy(
                    ei_hbm.at[1, pl.ds(crow + h * BCH, BCH), :], dst_v)

            for b in range(HALF):
                gather_start(b, b)
            for b in range(HALF, RING):
                gather_start(b, b)
                gather_wait(b - HALF, b - HALF)
                scatter_start(b - HALF, b - HALF)

            def group(g, _):
                j0 = RING + g * RING
                for b in range(RING):
                    j = j0 + b
                    scatter_wait(j - RING, b)
                    gather_start(j, b)
                    bm = (b - HALF) % RING
                    gather_wait(j - HALF, bm)
                    scatter_start(j - HALF, bm)
                return 0

            lax.fori_loop(0, GROUPS, group, 0)

            for i in range(PEEL):
                j = RING + GROUPS * RING + i
                b = j % RING
                scatter_wait(j - RING, b)
                gather_start(j, b)
                bm = (b - HALF) % RING
                gather_wait(j - HALF, bm)
                scatter_start(j - HALF, bm)

            for i in range(HALF):
                j = BCH - HALF + i
                b = j % RING
                gather_wait(j, b)
                scatter_start(j, b)
            for b in range(RING):
                jj = BCH - RING + b
                scatter_wait(jj, jj % RING)

        plsc.subcore_barrier()

        pltpu.sync_copy(agg_sh.at[pl.ds(rbase, ROWS_PT), :],
                        out_hbm.at[cid, pl.ds(rbase, ROWS_PT), :])
        pltpu.sync_copy(deg_sh.at[pl.ds(rbase, ROWS_PT), :],
                        outd_hbm.at[cid, pl.ds(rbase, ROWS_PT), :])

    return body(x, ei2, zrows, zdeg, ones8)


def _tc_xw1(x, W, b2d):
    R = 2000
    grid = (N_NODES // R,)

    def body(x_ref, w_ref, b_ref, o_ref):
        o_ref[...] = jnp.dot(
            x_ref[...], w_ref[:D], preferred_element_type=jnp.float32
        ) + b_ref[...]

    return pl.pallas_call(
        body,
        grid=grid,
        in_specs=[
            pl.BlockSpec((R, D), lambda i: (i, 0)),
            pl.BlockSpec((2 * D, D), lambda i: (0, 0)),
            pl.BlockSpec((1, D), lambda i: (0, 0)),
        ],
        out_specs=pl.BlockSpec((R, D), lambda i: (i, 0)),
        out_shape=jax.ShapeDtypeStruct((N_NODES, D), jnp.float32),
    )(x, W, b2d)


def _tc_combine(xw, part, dpart, W):
    R = 2000
    grid = (N_NODES // R,)

    def body(xw_ref, p_ref, d_ref, w_ref, o_ref):
        acc = p_ref[0] + p_ref[1]
        dsum = d_ref[0] + d_ref[1]
        deg = jnp.maximum(dsum[:, 0:1], 1.0)
        mean = acc / deg
        o_ref[...] = xw_ref[...] + jnp.dot(
            mean, w_ref[D:], preferred_element_type=jnp.float32)

    return pl.pallas_call(
        body,
        grid=grid,
        in_specs=[
            pl.BlockSpec((R, D), lambda i: (i, 0)),
            pl.BlockSpec((2, R, D), lambda i: (0, i, 0)),
            pl.BlockSpec((2, R, DG), lambda i: (0, i, 0)),
            pl.BlockSpec((2 * D, D), lambda i: (0, 0)),
        ],
        out_specs=pl.BlockSpec((R, D), lambda i: (i, 0)),
        out_shape=jax.ShapeDtypeStruct((N_NODES, D), jnp.float32),
    )(xw, part, dpart, W)


def kernel(x, edge_index, W, b):
    ei2 = edge_index.astype(jnp.int32).reshape(2, N_EDGES // K, K)
    zrows = jnp.zeros((ROWS_PT, D), jnp.float32)
    zdeg = jnp.zeros((ROWS_PT, DG), jnp.float32)
    ones8 = jnp.zeros((K, DG), jnp.float32).at[:, 0].set(1.0)
    xw = _tc_xw1(x, W, b.reshape(1, D))
    part, dpart = _sc_aggregate(x, ei2, zrows, zdeg, ones8)
    return _tc_combine(xw, part, dpart, W)
